# Initial kernel scaffold; baseline (speedup 1.0000x reference)
import jax
import jax.numpy as jnp
from jax.experimental import pallas as pl


def kernel(qkv):
    raise NotImplementedError("write your pallas kernel here")


# speedup vs baseline: 2.0421x; 2.0421x over previous
import functools

import jax
import jax.numpy as jnp
from jax.experimental import pallas as pl

WINDOW = 512
HEAD_DIM = 64
BQ = 512
NEG_INF = float("-inf")


def _attn_kernel(q_ref, k_ref, v_ref, o_ref, *, seq_len):
    scale = 1.0 / (HEAD_DIM ** 0.5)
    nq = seq_len // BQ
    for i in range(nq):
        qi = q_ref[0, i * BQ:(i + 1) * BQ, :] * scale
        if i == 0:
            ks = k_ref[0, 0:BQ, :]
            vs = v_ref[0, 0:BQ, :]
            q_idx = jax.lax.broadcasted_iota(jnp.int32, (BQ, BQ), 0)
            kv_idx = jax.lax.broadcasted_iota(jnp.int32, (BQ, BQ), 1)
            mask = q_idx >= kv_idx
        else:
            start = (i - 1) * BQ
            ks = k_ref[0, start:start + 2 * BQ, :]
            vs = v_ref[0, start:start + 2 * BQ, :]
            q_idx = i * BQ + jax.lax.broadcasted_iota(
                jnp.int32, (BQ, 2 * BQ), 0)
            kv_idx = start + jax.lax.broadcasted_iota(
                jnp.int32, (BQ, 2 * BQ), 1)
            diff = q_idx - kv_idx
            mask = (diff >= 0) & (diff <= WINDOW)
        scores = jax.lax.dot_general(
            qi, ks, (((1,), (1,)), ((), ())),
            preferred_element_type=jnp.float32)
        scores = jnp.where(mask, scores, NEG_INF)
        m = jnp.max(scores, axis=-1, keepdims=True)
        p = jnp.exp(scores - m)
        l = jnp.sum(p, axis=-1, keepdims=True)
        o = jax.lax.dot_general(
            p, vs, (((1,), (0,)), ((), ())),
            preferred_element_type=jnp.float32)
        o_ref[0, i * BQ:(i + 1) * BQ, :] = o / l


def kernel(qkv):
    b, l, _, h, e = qkv.shape
    qkv_t = jnp.transpose(qkv, (2, 0, 3, 1, 4))
    qkv_t = qkv_t.reshape(3, b * h, l, e)
    q, k, v = qkv_t[0], qkv_t[1], qkv_t[2]

    out = pl.pallas_call(
        functools.partial(_attn_kernel, seq_len=l),
        grid=(b * h,),
        in_specs=[
            pl.BlockSpec((1, l, e), lambda i: (i, 0, 0)),
            pl.BlockSpec((1, l, e), lambda i: (i, 0, 0)),
            pl.BlockSpec((1, l, e), lambda i: (i, 0, 0)),
        ],
        out_specs=pl.BlockSpec((1, l, e), lambda i: (i, 0, 0)),
        out_shape=jax.ShapeDtypeStruct((b * h, l, e), jnp.float32),
    )(q, k, v)

    return jnp.transpose(out.reshape(b, h, l, e), (0, 2, 1, 3))


# speedup vs baseline: 2.6522x; 1.2988x over previous
import jax
import jax.numpy as jnp
from jax.experimental import pallas as pl

WINDOW = 512
HEAD_DIM = 64
NUM_HEADS = 12
BQ = 512


def _attn_kernel(q_ref, km_ref, k_ref, vm_ref, v_ref, o_ref):
    i = pl.program_id(1)
    scale = 1.0 / (HEAD_DIM ** 0.5)
    q_idx = i * BQ + jax.lax.broadcasted_iota(jnp.int32, (BQ, 2 * BQ), 0)
    kv_idx = (i - 1) * BQ + jax.lax.broadcasted_iota(
        jnp.int32, (BQ, 2 * BQ), 1)
    diff = q_idx - kv_idx
    mask = (diff >= 0) & (diff <= WINDOW) & (kv_idx >= 0)
    neg_inf = jnp.float32(float("-inf"))
    for hh in range(NUM_HEADS):
        c0 = hh * HEAD_DIM
        qh = q_ref[0, :, c0:c0 + HEAD_DIM] * scale
        kh = jnp.concatenate(
            [km_ref[0, :, c0:c0 + HEAD_DIM], k_ref[0, :, c0:c0 + HEAD_DIM]],
            axis=0)
        vh = jnp.concatenate(
            [vm_ref[0, :, c0:c0 + HEAD_DIM], v_ref[0, :, c0:c0 + HEAD_DIM]],
            axis=0)
        s = jax.lax.dot_general(
            qh, kh, (((1,), (1,)), ((), ())),
            preferred_element_type=jnp.float32)
        s = jnp.where(mask, s, neg_inf)
        m = jnp.max(s, axis=-1, keepdims=True)
        p = jnp.exp(s - m)
        denom = jnp.sum(p, axis=-1, keepdims=True)
        oh = jax.lax.dot_general(
            p, vh, (((1,), (0,)), ((), ())),
            preferred_element_type=jnp.float32)
        o_ref[0, :, c0:c0 + HEAD_DIM] = oh / denom


def kernel(qkv):
    b, l, three, h, e = qkv.shape
    ch = h * e
    x = qkv.reshape(b, l, three * ch)
    nq = l // BQ

    blk = (1, BQ, ch)
    out = pl.pallas_call(
        _attn_kernel,
        grid=(b, nq),
        in_specs=[
            pl.BlockSpec(blk, lambda ib, i: (ib, i, 0)),
            pl.BlockSpec(blk, lambda ib, i: (ib, jnp.maximum(i - 1, 0), 1)),
            pl.BlockSpec(blk, lambda ib, i: (ib, i, 1)),
            pl.BlockSpec(blk, lambda ib, i: (ib, jnp.maximum(i - 1, 0), 2)),
            pl.BlockSpec(blk, lambda ib, i: (ib, i, 2)),
        ],
        out_specs=pl.BlockSpec(blk, lambda ib, i: (ib, i, 0)),
        out_shape=jax.ShapeDtypeStruct((b, l, ch), jnp.float32),
    )(x, x, x, x, x)

    return out.reshape(b, l, h, e)


# speedup vs baseline: 3.0693x; 1.1573x over previous
import jax
import jax.numpy as jnp
from jax.experimental import pallas as pl

WINDOW = 512
HEAD_DIM = 64
NUM_HEADS = 12
BQ = 256
KB = 3 * BQ


def _attn_kernel(q_ref, km2_ref, km1_ref, kc_ref, vm2_ref, vm1_ref, vc_ref,
                 o_ref):
    i = pl.program_id(1)
    scale = 1.0 / (HEAD_DIM ** 0.5)
    q_idx = i * BQ + jax.lax.broadcasted_iota(jnp.int32, (BQ, KB), 0)
    kv_idx = (i - 2) * BQ + jax.lax.broadcasted_iota(jnp.int32, (BQ, KB), 1)
    diff = q_idx - kv_idx
    mask = (diff >= 0) & (diff <= WINDOW) & (kv_idx >= 0)
    bias = jnp.where(mask, jnp.float32(0), jnp.float32(float("-inf")))
    for hh in range(NUM_HEADS):
        c0 = hh * HEAD_DIM
        qh = q_ref[0, :, c0:c0 + HEAD_DIM] * scale
        kh = jnp.concatenate(
            [km2_ref[0, :, c0:c0 + HEAD_DIM],
             km1_ref[0, :, c0:c0 + HEAD_DIM],
             kc_ref[0, :, c0:c0 + HEAD_DIM]], axis=0)
        vh = jnp.concatenate(
            [vm2_ref[0, :, c0:c0 + HEAD_DIM],
             vm1_ref[0, :, c0:c0 + HEAD_DIM],
             vc_ref[0, :, c0:c0 + HEAD_DIM]], axis=0)
        s = jax.lax.dot_general(
            qh, kh, (((1,), (1,)), ((), ())),
            preferred_element_type=jnp.float32) + bias
        m = jnp.max(s, axis=-1, keepdims=True)
        p = jnp.exp(s - m)
        denom = jnp.sum(p, axis=-1, keepdims=True)
        oh = jax.lax.dot_general(
            p, vh, (((1,), (0,)), ((), ())),
            preferred_element_type=jnp.float32)
        o_ref[0, :, c0:c0 + HEAD_DIM] = oh * (1.0 / denom)


def kernel(qkv):
    b, l, three, h, e = qkv.shape
    ch = h * e
    x = qkv.reshape(b, l, three * ch)
    nq = l // BQ

    blk = (1, BQ, ch)
    out = pl.pallas_call(
        _attn_kernel,
        grid=(b, nq),
        in_specs=[
            pl.BlockSpec(blk, lambda ib, i: (ib, i, 0)),
            pl.BlockSpec(blk, lambda ib, i: (ib, jnp.maximum(i - 2, 0), 1)),
            pl.BlockSpec(blk, lambda ib, i: (ib, jnp.maximum(i - 1, 0), 1)),
            pl.BlockSpec(blk, lambda ib, i: (ib, i, 1)),
            pl.BlockSpec(blk, lambda ib, i: (ib, jnp.maximum(i - 2, 0), 2)),
            pl.BlockSpec(blk, lambda ib, i: (ib, jnp.maximum(i - 1, 0), 2)),
            pl.BlockSpec(blk, lambda ib, i: (ib, i, 2)),
        ],
        out_specs=pl.BlockSpec(blk, lambda ib, i: (ib, i, 0)),
        out_shape=jax.ShapeDtypeStruct((b, l, ch), jnp.float32),
    )(x, x, x, x, x, x, x)

    return out.reshape(b, l, h, e)
